# baseline (speedup 1.0000x reference)
import functools

import jax
import jax.numpy as jnp
from jax import lax
from jax.experimental import pallas as pl
from jax.experimental.pallas import tpu as pltpu
from jax.experimental.pallas import tpu_sc as plsc

T = 512
CH = 3
H = 224
W = 224
NKEY = 171
NW = 32
NS = CH * H
SPT = NS // NW
NCH = 4
CW = W // NCH
FB = 11
GRP = 8


def kernel(video):
    v3 = jnp.transpose(video, (1, 2, 3, 0)).reshape(NS, W, T)
    mesh = plsc.VectorSubcoreMesh(core_axis_name="c", subcore_axis_name="s")

    @functools.partial(
        pl.kernel,
        mesh=mesh,
        out_type=jax.ShapeDtypeStruct((NKEY, CH, H, W), jnp.float32),
        scratch_types=(
            [pltpu.VMEM((CW, T), jnp.float32)] * 2
            + [pltpu.VMEM((NKEY, W), jnp.float32)]
            + [pltpu.SemaphoreType.DMA] * 3
        ),
        compiler_params=pltpu.CompilerParams(
            use_tc_tiling_on_sc=True, needs_layout_passes=False),
    )
    def k(v_hbm, o_hbm, ib0, ib1, ob, *sems):
        inbufs = (ib0, ib1)
        gsems = sems[0:2]
        ssem = sems[2]
        wid = lax.axis_index("s") * 2 + lax.axis_index("c")
        w16 = lax.iota(jnp.int32, 16)

        def in_copy(u, k_):
            return pltpu.make_async_copy(
                v_hbm.at[u * NW + wid, pl.ds(k_ * CW, CW)],
                inbufs[k_ % 2],
                gsems[k_ % 2],
            )

        def out_copy(u):
            s = u * NW + wid
            return pltpu.make_async_copy(
                ob, o_hbm.at[:, s // H, s % H], ssem)

        def do_stripe(u, prefetch_next):
            for k_ in range(NCH):
                if k_ + 1 < NCH:
                    in_copy(u, k_ + 1).start()
                elif prefetch_next:
                    in_copy(u + 1, 0).start()
                if k_ == 0:
                    @pl.when(u >= 1)
                    def _():
                        out_copy(u - 1).wait()
                in_copy(u, k_).wait()
                inb = inbufs[k_ % 2]

                def fblk(b, _):
                    fv = b * 16 + w16
                    srcv = jnp.clip(3 * fv - 1, 0, T - 1)
                    fmask = fv < NKEY
                    fvc = jnp.minimum(fv, NKEY - 1)
                    for w0 in range(0, CW, GRP):
                        vs = []
                        for d in range(GRP):
                            wv = jnp.full((16,), w0 + d, jnp.int32)
                            vs.append(plsc.load_gather(inb, [wv, srcv]))
                        for d in range(GRP):
                            cv = jnp.full((16,), k_ * CW + w0 + d, jnp.int32)
                            plsc.store_scatter(
                                ob, [fvc, cv], vs[d], mask=fmask)
                    return 0

                lax.fori_loop(0, FB, fblk, 0)
            out_copy(u).start()

        in_copy(0, 0).start()

        def stripe_body(u, _):
            do_stripe(u, True)
            return 0

        lax.fori_loop(0, SPT - 1, stripe_body, 0)
        do_stripe(SPT - 1, False)
        out_copy(SPT - 1).wait()

    return k(v3)


# speedup vs baseline: 1.7460x; 1.7460x over previous
import functools

import jax
import jax.numpy as jnp
from jax import lax
from jax.experimental import pallas as pl
from jax.experimental.pallas import tpu as pltpu
from jax.experimental.pallas import tpu_sc as plsc

T = 512
CH = 3
H = 224
W = 224
NKEY = 171
NW = 32
NS = CH * H
SPT = NS // NW
NCH = 14
CW = W // NCH
FB = 11
GRP = 8


def kernel(video):
    v3 = jnp.transpose(video, (1, 2, 3, 0)).reshape(NS, W, T)
    mesh = plsc.VectorSubcoreMesh(core_axis_name="c", subcore_axis_name="s")

    @functools.partial(
        pl.kernel,
        mesh=mesh,
        out_type=jax.ShapeDtypeStruct((NKEY, CH, H, W), jnp.float32),
        scratch_types=(
            [pltpu.VMEM((CW, T), jnp.float32)] * 2
            + [pltpu.VMEM((NKEY, W), jnp.float32)]
            + [pltpu.SemaphoreType.DMA] * 3
        ),
        compiler_params=pltpu.CompilerParams(
            use_tc_tiling_on_sc=True, needs_layout_passes=False),
    )
    def k(v_hbm, o_hbm, ib0, ib1, ob, *sems):
        inbufs = (ib0, ib1)
        gsems = sems[0:2]
        ssem = sems[2]
        wid = lax.axis_index("s") * 2 + lax.axis_index("c")
        w16 = lax.iota(jnp.int32, 16)

        def in_copy(u, k_):
            return pltpu.make_async_copy(
                v_hbm.at[u * NW + wid, pl.ds(k_ * CW, CW)],
                inbufs[k_ % 2],
                gsems[k_ % 2],
            )

        def out_copy(u):
            s = u * NW + wid
            return pltpu.make_async_copy(
                ob, o_hbm.at[:, s // H, s % H], ssem)

        def do_stripe(u, prefetch_next):
            for k_ in range(NCH):
                if k_ + 1 < NCH:
                    in_copy(u, k_ + 1).start()
                elif prefetch_next:
                    in_copy(u + 1, 0).start()
                if k_ == 0:
                    @pl.when(u >= 1)
                    def _():
                        out_copy(u - 1).wait()
                in_copy(u, k_).wait()
                inb = inbufs[k_ % 2]

                def fblk(b, _):
                    fv = b * 16 + w16
                    srcv = jnp.clip(3 * fv - 1, 0, T - 1)
                    fmask = fv < NKEY
                    fvc = jnp.minimum(fv, NKEY - 1)
                    for d0 in range(0, 16, GRP):
                        vs = []
                        for d in range(d0, d0 + GRP):
                            wd = (w16 + d) & 15
                            vs.append(plsc.load_gather(inb, [wd, srcv]))
                        for d in range(d0, d0 + GRP):
                            wd = (w16 + d) & 15
                            plsc.store_scatter(
                                ob, [fvc, k_ * CW + wd], vs[d - d0],
                                mask=fmask)
                    return 0

                lax.fori_loop(0, FB, fblk, 0)
            out_copy(u).start()

        in_copy(0, 0).start()

        def stripe_body(u, _):
            do_stripe(u, True)
            return 0

        lax.fori_loop(0, SPT - 1, stripe_body, 0)
        do_stripe(SPT - 1, False)
        out_copy(SPT - 1).wait()

    return k(v3)


# speedup vs baseline: 2.3968x; 1.3727x over previous
import functools

import jax
import jax.numpy as jnp
from jax import lax
from jax.experimental import pallas as pl
from jax.experimental.pallas import tpu as pltpu
from jax.experimental.pallas import tpu_sc as plsc

T = 512
CH = 3
H = 224
W = 224
NKEY = 171
NW = 32
NS = CH * H
SPT = NS // NW
NCH = 14
CW = W // NCH
FB = 11
GRP = 8


def kernel(video):
    v3 = jnp.transpose(video, (1, 2, 3, 0)).reshape(NS, W, T)
    mesh = plsc.VectorSubcoreMesh(core_axis_name="c", subcore_axis_name="s")

    @functools.partial(
        pl.kernel,
        mesh=mesh,
        out_type=jax.ShapeDtypeStruct((NKEY, CH, H, W), jnp.float32),
        scratch_types=(
            [pltpu.VMEM((CW, T), jnp.float32)] * 4
            + [pltpu.VMEM((NKEY, W), jnp.float32)] * 2
            + [pltpu.SemaphoreType.DMA] * 6
        ),
        compiler_params=pltpu.CompilerParams(
            use_tc_tiling_on_sc=True, needs_layout_passes=False),
    )
    def k(v_hbm, o_hbm, ib0, ib1, ib2, ib3, ob0, ob1, *sems):
        inbufs = (ib0, ib1, ib2, ib3)
        outbufs = (ob0, ob1)
        gsems = sems[0:4]
        ssems = sems[4:6]
        wid = lax.axis_index("s") * 2 + lax.axis_index("c")
        w16 = lax.iota(jnp.int32, 16)

        def in_copy(u, up, k_):
            b = (2 * up + k_) % 4
            return pltpu.make_async_copy(
                v_hbm.at[u * NW + wid, pl.ds(k_ * CW, CW)],
                inbufs[b],
                gsems[b],
            )

        def out_copy(u, up):
            s = u * NW + wid
            return pltpu.make_async_copy(
                outbufs[up], o_hbm.at[:, s // H, s % H], ssems[up])

        def do_stripe(u, up, prefetch_next):
            ob = outbufs[up]
            for k_ in range(NCH):
                if k_ + 3 < NCH:
                    in_copy(u, up, k_ + 3).start()
                elif prefetch_next:
                    in_copy(u + 1, 1 - up, k_ + 3 - NCH).start()
                if k_ == 0:
                    @pl.when(u >= 2)
                    def _():
                        out_copy(u - 2, up).wait()
                in_copy(u, up, k_).wait()
                inb = inbufs[(2 * up + k_) % 4]

                def fblk(b, _):
                    fv = b * 16 + w16
                    srcv = jnp.clip(3 * fv - 1, 0, T - 1)
                    fmask = fv < NKEY
                    fvc = jnp.minimum(fv, NKEY - 1)
                    for d0 in range(0, 16, GRP):
                        vs = []
                        for d in range(d0, d0 + GRP):
                            wd = (w16 + d) & 15
                            vs.append(plsc.load_gather(inb, [wd, srcv]))
                        for d in range(d0, d0 + GRP):
                            wd = (w16 + d) & 15
                            plsc.store_scatter(
                                ob, [fvc, k_ * CW + wd], vs[d - d0],
                                mask=fmask)
                    return 0

                lax.fori_loop(0, FB, fblk, 0)
            out_copy(u, up).start()

        for kp in range(3):
            in_copy(0, 0, kp).start()

        def pair_body(pr, _):
            u = pr * 2
            do_stripe(u, 0, True)
            do_stripe(u + 1, 1, True)
            return 0

        lax.fori_loop(0, SPT // 2, pair_body, 0)
        do_stripe(SPT - 1, 0, False)
        for u, up in ((SPT - 2, 1), (SPT - 1, 0)):
            out_copy(u, up).wait()

    return k(v3)
